# baseline (speedup 1.0000x reference)
import functools

import jax
import jax.numpy as jnp
from jax import lax
from jax.experimental import pallas as pl
from jax.experimental.pallas import tpu as pltpu
from jax.experimental.pallas import tpu_sc as plsc

HE = 512
NSRC = 512
F = 128
NB = 4
NEG = -1e30

NNZ = 16384
NTILES = 32
CHUNK = NNZ // NTILES
BINS = HE * NSRC
SLICE = BINS // 16
ZCHUNK = 2048


def _hist_body(src_hbm, dst_hbm, out_hbm, sv, dv, idxv, onesv, zbuf, shared,
               sem1, sem2):
    c = lax.axis_index("c")
    s = lax.axis_index("s")
    wid = s * 2 + c
    base = wid * CHUNK
    ld1 = pltpu.async_copy(src_hbm.at[pl.ds(base, CHUNK)], sv, sem1)
    ld2 = pltpu.async_copy(dst_hbm.at[pl.ds(base, CHUNK)], dv, sem2)

    for k in range(ZCHUNK // 16):
        zbuf[pl.ds(k * 16, 16)] = jnp.zeros((16,), jnp.float32)
    for q in range(SLICE // ZCHUNK):
        pltpu.sync_copy(zbuf, shared.at[pl.ds(s * SLICE + q * ZCHUNK, ZCHUNK)])
    for k in range(8):
        onesv[pl.ds(k * 16, 16)] = jnp.ones((16,), jnp.float32)
    ld1.wait()
    ld2.wait()
    for k in range(CHUNK // 16):
        srcv = sv[pl.ds(k * 16, 16)]
        dstv = dv[pl.ds(k * 16, 16)]
        f = ((srcv >> 7) << 16) + dstv * 128 + (srcv & 127)
        idxv[k // 8, pl.ds((k % 8) * 16, 16)] = f
    plsc.subcore_barrier()
    cps = [pltpu.async_copy(onesv, shared.at[idxv.at[j]], sem1, add=True)
           for j in range(CHUNK // 128)]
    for cp in cps:
        cp.wait()
    plsc.subcore_barrier()
    off = c * BINS + s * SLICE
    pltpu.sync_copy(shared.at[pl.ds(s * SLICE, SLICE)],
                    out_hbm.at[pl.ds(off, SLICE)])


def _hist_sc(src, dst):
    run = functools.partial(
        pl.kernel,
        mesh=plsc.VectorSubcoreMesh(core_axis_name="c", subcore_axis_name="s"),
        out_type=jax.ShapeDtypeStruct((2 * BINS,), jnp.float32),
        scratch_types=[
            pltpu.VMEM((CHUNK,), jnp.int32),
            pltpu.VMEM((CHUNK,), jnp.int32),
            pltpu.VMEM((CHUNK // 128, 128), jnp.int32),
            pltpu.VMEM((128,), jnp.float32),
            pltpu.VMEM((ZCHUNK,), jnp.float32),
            pltpu.VMEM_SHARED((BINS,), jnp.float32),
            pltpu.SemaphoreType.DMA,
            pltpu.SemaphoreType.DMA,
        ],
    )(_hist_body)
    return run(src, dst)


def _dense_body(x_ref, hist_ref, w_ref, att_ref, out_ref, loss_ref,
                zb_ref, ob_ref, zsem, osem):
    nb = x_ref.shape[0]
    zb_ref[...] = jnp.zeros_like(zb_ref)
    zcps = [pltpu.async_copy(zb_ref, out_ref.at[b, pl.ds(NSRC, 2048 - NSRC), :],
                             zsem) for b in range(nb)]
    xw4 = jnp.dot(x_ref[...].reshape(nb * NSRC, F), w_ref[:],
                  preferred_element_type=jnp.float32)
    Hc = hist_ref[0] + hist_ref[1]
    Hj = [Hc[j * HE:(j + 1) * HE, :] for j in range(NB)]
    maskj = [h > 0 for h in Hj]
    deg_e = jnp.zeros((HE, 1), jnp.float32)
    for j in range(NB):
        deg_e = deg_e + jnp.sum(Hj[j], axis=1, keepdims=True)
    Bn = jnp.where(deg_e > 0, 1.0 / deg_e, 0.0)
    ones_col = jnp.ones((HE, 1), jnp.float32)
    Dj = [lax.dot_general(h, ones_col, (((0,), (0,)), ((), ())),
                          preferred_element_type=jnp.float32)
          for h in Hj]

    attv = att_ref[0]
    a1 = attv[:, :F]
    a2 = attv[:, F:]

    acc = None
    sum_i = 0.0
    sum_j = 0.0
    ocps = []
    for b in range(nb):
        xwj = [xw4[b * NSRC + j * F:b * NSRC + (j + 1) * F, :]
               for j in range(NB)]
        es = jnp.zeros((HE, F), jnp.float32)
        for j in range(NB):
            es = es + jnp.dot(Hj[j], xwj[j], preferred_element_type=jnp.float32)
        z1 = lax.dot_general(es, a2, (((1,), (1,)), ((), ())),
                             preferred_element_type=jnp.float32)
        out_e = jnp.zeros((HE, F), jnp.float32)
        Sj = []
        for j in range(NB):
            y1 = lax.dot_general(a1, xwj[j], (((1,), (1,)), ((), ())),
                                 preferred_element_type=jnp.float32)
            L = z1 + y1
            L = jnp.where(L >= 0, L, 0.2 * L)
            Lm = jnp.where(maskj[j], L, NEG)
            mx = jnp.max(Lm, axis=0, keepdims=True)
            mx = jnp.where(mx > 0.5 * NEG, mx, 0.0)
            CE = Hj[j] * jnp.exp(Lm - mx)
            s = jnp.sum(CE, axis=0, keepdims=True)
            S = CE / (s + 1e-16)
            Sj.append(S)
            out_e = out_e + jnp.dot(S, xwj[j], preferred_element_type=jnp.float32)
        out_e = Bn * out_e
        for j in range(NB):
            out_n = Dj[j] * lax.dot_general(
                Sj[j], out_e, (((0,), (0,)), ((), ())),
                preferred_element_type=jnp.float32)
            ob_ref[b, j * F:(j + 1) * F, :] = out_n
            sum_i = sum_i + jnp.sum(Dj[j] * jnp.sum(xwj[j], axis=1,
                                                    keepdims=True))
        ocps.append(pltpu.async_copy(ob_ref.at[b],
                                     out_ref.at[b, pl.ds(0, NSRC), :], osem))
        sum_j = sum_j + jnp.sum(deg_e * jnp.sum(es, axis=1, keepdims=True))

        inner = lax.dot_general(es, es, (((1,), (1,)), ((), ())),
                                preferred_element_type=jnp.float32)
        sq = jnp.sum(es * es, axis=1, keepdims=True)
        norms = jnp.sqrt(sq)
        alpha = inner / (norms * jnp.transpose(norms) + 1e-16)
        d2 = jnp.clip(sq + jnp.transpose(sq) - 2.0 * inner, 0.0, None)
        dist = jnp.sqrt(d2 + 1e-12)
        li = alpha * dist + (1.0 - alpha) * jnp.clip(4.2 - dist, 0.0, None)
        acc = li if acc is None else acc + li

    mkm = acc * (1.0 / nb)
    row_id = lax.broadcasted_iota(jnp.int32, (HE, 1), 0).astype(jnp.float32)
    num_he = jnp.max(jnp.where(deg_e > 0, row_id + 1.0, 0.0))
    rk = lax.broadcasted_iota(jnp.int32, (HE, HE), 0).astype(jnp.float32)
    rm = lax.broadcasted_iota(jnp.int32, (HE, HE), 1).astype(jnp.float32)
    mask_km = jnp.where((rk < num_he) & (rm < num_he), 1.0, 0.0)
    loss_hyper = jnp.sum(jnp.abs(mkm) * mask_km) / (num_he + 1.0) ** 2
    total = float(NNZ) * nb * F
    loss_ref[0, 0] = jnp.abs((sum_i - sum_j) / total) + loss_hyper
    for cp in zcps:
        cp.wait()
    for cp in ocps:
        cp.wait()


def _dense_call(x, hist, weight, att):
    B, N, _ = x.shape
    return pl.pallas_call(
        _dense_body,
        grid=(1,),
        in_specs=[
            pl.BlockSpec((B, NSRC, F), lambda b: (0, 0, 0)),
            pl.BlockSpec((2, NB * HE, F), lambda b: (0, 0, 0)),
            pl.BlockSpec((F, F), lambda b: (0, 0)),
            pl.BlockSpec((1, 1, 2 * F), lambda b: (0, 0, 0)),
        ],
        out_specs=[
            pl.BlockSpec(memory_space=pl.ANY),
            pl.BlockSpec(memory_space=pltpu.SMEM, block_shape=(1, 1),
                         index_map=lambda b: (0, 0)),
        ],
        out_shape=[
            jax.ShapeDtypeStruct((B, N, F), jnp.float32),
            jax.ShapeDtypeStruct((1, 1), jnp.float32),
        ],
        scratch_shapes=[
            pltpu.VMEM((2048 - NSRC, F), jnp.float32),
            pltpu.VMEM((B, NSRC, F), jnp.float32),
            pltpu.SemaphoreType.DMA,
            pltpu.SemaphoreType.DMA,
        ],
    )(x, hist, weight, att)


def kernel(x, hyperedge_index, weight, att):
    src = hyperedge_index[0]
    dst = hyperedge_index[1]
    hist = _hist_sc(src, dst).reshape(2, NB * HE, F)
    out, loss = _dense_call(x, hist, weight, att)
    return out, loss[0, 0]


# speedup vs baseline: 1.0110x; 1.0110x over previous
import functools

import jax
import jax.numpy as jnp
from jax import lax
from jax.experimental import pallas as pl
from jax.experimental.pallas import tpu as pltpu
from jax.experimental.pallas import tpu_sc as plsc

HE = 512
NSRC = 512
F = 128
NB = 4
NEG = -1e30

NNZ = 16384
NTILES = 32
CHUNK = NNZ // NTILES
BINS = HE * NSRC
SLICE = BINS // 16
ZCHUNK = 2048


def _hist_body(src_hbm, dst_hbm, out_hbm, sv, dv, idxv, onesv, zbuf, shared,
               sem1, sem2, sem3):
    c = lax.axis_index("c")
    s = lax.axis_index("s")
    wid = s * 2 + c
    base = wid * CHUNK
    ld1 = pltpu.async_copy(src_hbm.at[pl.ds(base, CHUNK)], sv, sem1)
    ld2 = pltpu.async_copy(dst_hbm.at[pl.ds(base, CHUNK)], dv, sem2)

    for k in range(ZCHUNK // 16):
        zbuf[pl.ds(k * 16, 16)] = jnp.zeros((16,), jnp.float32)
    zcps = [pltpu.async_copy(
        zbuf, shared.at[pl.ds(s * SLICE + q * ZCHUNK, ZCHUNK)], sem3)
        for q in range(SLICE // ZCHUNK)]
    for k in range(8):
        onesv[pl.ds(k * 16, 16)] = jnp.ones((16,), jnp.float32)
    ld1.wait()
    ld2.wait()
    for k in range(CHUNK // 16):
        srcv = sv[pl.ds(k * 16, 16)]
        dstv = dv[pl.ds(k * 16, 16)]
        f = ((srcv >> 7) << 16) + dstv * 128 + (srcv & 127)
        idxv[k // 8, pl.ds((k % 8) * 16, 16)] = f
    for cp in zcps:
        cp.wait()
    plsc.subcore_barrier()
    cps = [pltpu.async_copy(onesv, shared.at[idxv.at[j]], sem1, add=True)
           for j in range(CHUNK // 128)]
    for cp in cps:
        cp.wait()
    plsc.subcore_barrier()
    off = c * BINS + s * SLICE
    pltpu.sync_copy(shared.at[pl.ds(s * SLICE, SLICE)],
                    out_hbm.at[pl.ds(off, SLICE)])


def _hist_sc(src, dst):
    run = functools.partial(
        pl.kernel,
        mesh=plsc.VectorSubcoreMesh(core_axis_name="c", subcore_axis_name="s"),
        out_type=jax.ShapeDtypeStruct((2 * BINS,), jnp.float32),
        scratch_types=[
            pltpu.VMEM((CHUNK,), jnp.int32),
            pltpu.VMEM((CHUNK,), jnp.int32),
            pltpu.VMEM((CHUNK // 128, 128), jnp.int32),
            pltpu.VMEM((128,), jnp.float32),
            pltpu.VMEM((ZCHUNK,), jnp.float32),
            pltpu.VMEM_SHARED((BINS,), jnp.float32),
            pltpu.SemaphoreType.DMA,
            pltpu.SemaphoreType.DMA,
            pltpu.SemaphoreType.DMA,
        ],
    )(_hist_body)
    return run(src, dst)


def _dense_body(x_ref, hist_ref, w_ref, att_ref, out_ref, loss_ref):
    nb = x_ref.shape[0]
    xw4 = jnp.dot(x_ref[...].reshape(nb * NSRC, F), w_ref[:],
                  preferred_element_type=jnp.float32)
    Hc = hist_ref[0] + hist_ref[1]
    Hj = [Hc[j * HE:(j + 1) * HE, :] for j in range(NB)]
    maskj = [h > 0 for h in Hj]
    deg_e = jnp.zeros((HE, 1), jnp.float32)
    for j in range(NB):
        deg_e = deg_e + jnp.sum(Hj[j], axis=1, keepdims=True)
    Bn = jnp.where(deg_e > 0, 1.0 / deg_e, 0.0)
    ones_col = jnp.ones((HE, 1), jnp.float32)
    Dj = [lax.dot_general(h, ones_col, (((0,), (0,)), ((), ())),
                          preferred_element_type=jnp.float32)
          for h in Hj]

    attv = att_ref[0]
    a1 = attv[:, :F]
    a2 = attv[:, F:]

    acc = None
    sum_i = 0.0
    sum_j = 0.0
    for b in range(nb):
        xwj = [xw4[b * NSRC + j * F:b * NSRC + (j + 1) * F, :]
               for j in range(NB)]
        es = jnp.zeros((HE, F), jnp.float32)
        for j in range(NB):
            es = es + jnp.dot(Hj[j], xwj[j], preferred_element_type=jnp.float32)
        z1 = lax.dot_general(es, a2, (((1,), (1,)), ((), ())),
                             preferred_element_type=jnp.float32)
        out_e = jnp.zeros((HE, F), jnp.float32)
        Sj = []
        for j in range(NB):
            y1 = lax.dot_general(a1, xwj[j], (((1,), (1,)), ((), ())),
                                 preferred_element_type=jnp.float32)
            L = z1 + y1
            L = jnp.where(L >= 0, L, 0.2 * L)
            Lm = jnp.where(maskj[j], L, NEG)
            mx = jnp.max(Lm, axis=0, keepdims=True)
            mx = jnp.where(mx > 0.5 * NEG, mx, 0.0)
            CE = Hj[j] * jnp.exp(Lm - mx)
            s = jnp.sum(CE, axis=0, keepdims=True)
            S = CE / (s + 1e-16)
            Sj.append(S)
            out_e = out_e + jnp.dot(S, xwj[j], preferred_element_type=jnp.float32)
        out_e = Bn * out_e
        for j in range(NB):
            out_n = Dj[j] * lax.dot_general(
                Sj[j], out_e, (((0,), (0,)), ((), ())),
                preferred_element_type=jnp.float32)
            out_ref[b, j * F:(j + 1) * F, :] = out_n
            sum_i = sum_i + jnp.sum(Dj[j] * jnp.sum(xwj[j], axis=1,
                                                    keepdims=True))
        sum_j = sum_j + jnp.sum(deg_e * jnp.sum(es, axis=1, keepdims=True))

        inner = lax.dot_general(es, es, (((1,), (1,)), ((), ())),
                                preferred_element_type=jnp.float32)
        sq = jnp.sum(es * es, axis=1, keepdims=True)
        norms = jnp.sqrt(sq)
        alpha = inner / (norms * jnp.transpose(norms) + 1e-16)
        d2 = jnp.clip(sq + jnp.transpose(sq) - 2.0 * inner, 0.0, None)
        dist = jnp.sqrt(d2 + 1e-12)
        li = alpha * dist + (1.0 - alpha) * jnp.clip(4.2 - dist, 0.0, None)
        acc = li if acc is None else acc + li

    out_ref[:, NSRC:, :] = jnp.zeros_like(out_ref[:, NSRC:, :])

    mkm = acc * (1.0 / nb)
    row_id = lax.broadcasted_iota(jnp.int32, (HE, 1), 0).astype(jnp.float32)
    num_he = jnp.max(jnp.where(deg_e > 0, row_id + 1.0, 0.0))
    rk = lax.broadcasted_iota(jnp.int32, (HE, HE), 0).astype(jnp.float32)
    rm = lax.broadcasted_iota(jnp.int32, (HE, HE), 1).astype(jnp.float32)
    mask_km = jnp.where((rk < num_he) & (rm < num_he), 1.0, 0.0)
    loss_hyper = jnp.sum(jnp.abs(mkm) * mask_km) / (num_he + 1.0) ** 2
    total = float(NNZ) * nb * F
    loss_ref[0, 0] = jnp.abs((sum_i - sum_j) / total) + loss_hyper


def _dense_call(x, hist, weight, att):
    B, N, _ = x.shape
    return pl.pallas_call(
        _dense_body,
        grid=(1,),
        in_specs=[
            pl.BlockSpec((B, NSRC, F), lambda b: (0, 0, 0)),
            pl.BlockSpec((2, NB * HE, F), lambda b: (0, 0, 0)),
            pl.BlockSpec((F, F), lambda b: (0, 0)),
            pl.BlockSpec((1, 1, 2 * F), lambda b: (0, 0, 0)),
        ],
        out_specs=[
            pl.BlockSpec((B, N, F), lambda b: (0, 0, 0)),
            pl.BlockSpec(memory_space=pltpu.SMEM, block_shape=(1, 1),
                         index_map=lambda b: (0, 0)),
        ],
        out_shape=[
            jax.ShapeDtypeStruct((B, N, F), jnp.float32),
            jax.ShapeDtypeStruct((1, 1), jnp.float32),
        ],
    )(x, hist, weight, att)


def kernel(x, hyperedge_index, weight, att):
    src = hyperedge_index[0]
    dst = hyperedge_index[1]
    hist = _hist_sc(src, dst).reshape(2, NB * HE, F)
    out, loss = _dense_call(x, hist, weight, att)
    return out, loss[0, 0]


# speedup vs baseline: 1.0223x; 1.0112x over previous
import functools

import jax
import jax.numpy as jnp
from jax import lax
from jax.experimental import pallas as pl
from jax.experimental.pallas import tpu as pltpu
from jax.experimental.pallas import tpu_sc as plsc

HE = 512
NSRC = 512
F = 128
NB = 4
NEG = -1e30

NNZ = 16384
NTILES = 32
CHUNK = NNZ // NTILES
BINS = HE * NSRC
SLICE = BINS // 16
ZCHUNK = 2048


def _hist_body(he_hbm, out_hbm, sv, dv, idxv, onesv, zbuf, shared,
               sem1, sem2, sem3):
    c = lax.axis_index("c")
    s = lax.axis_index("s")
    wid = s * 2 + c
    base = wid * CHUNK
    ld1 = pltpu.async_copy(he_hbm.at[0, pl.ds(base, CHUNK)], sv, sem1)
    ld2 = pltpu.async_copy(he_hbm.at[1, pl.ds(base, CHUNK)], dv, sem2)

    for k in range(ZCHUNK // 16):
        zbuf[pl.ds(k * 16, 16)] = jnp.zeros((16,), jnp.float32)
    zcps = [pltpu.async_copy(
        zbuf, shared.at[pl.ds(s * SLICE + q * ZCHUNK, ZCHUNK)], sem3)
        for q in range(SLICE // ZCHUNK)]
    for k in range(8):
        onesv[pl.ds(k * 16, 16)] = jnp.ones((16,), jnp.float32)
    ld1.wait()
    ld2.wait()
    for k in range(CHUNK // 16):
        srcv = sv[pl.ds(k * 16, 16)]
        dstv = dv[pl.ds(k * 16, 16)]
        f = ((srcv >> 7) << 16) + dstv * 128 + (srcv & 127)
        idxv[k // 8, pl.ds((k % 8) * 16, 16)] = f
    for cp in zcps:
        cp.wait()
    plsc.subcore_barrier()
    cps = [pltpu.async_copy(onesv, shared.at[idxv.at[j]], sem1, add=True)
           for j in range(CHUNK // 128)]
    for cp in cps:
        cp.wait()
    plsc.subcore_barrier()
    off = c * BINS + s * SLICE
    pltpu.sync_copy(shared.at[pl.ds(s * SLICE, SLICE)],
                    out_hbm.at[pl.ds(off, SLICE)])


def _hist_sc(he):
    run = functools.partial(
        pl.kernel,
        mesh=plsc.VectorSubcoreMesh(core_axis_name="c", subcore_axis_name="s"),
        out_type=jax.ShapeDtypeStruct((2 * BINS,), jnp.float32),
        scratch_types=[
            pltpu.VMEM((CHUNK,), jnp.int32),
            pltpu.VMEM((CHUNK,), jnp.int32),
            pltpu.VMEM((CHUNK // 128, 128), jnp.int32),
            pltpu.VMEM((128,), jnp.float32),
            pltpu.VMEM((ZCHUNK,), jnp.float32),
            pltpu.VMEM_SHARED((BINS,), jnp.float32),
            pltpu.SemaphoreType.DMA,
            pltpu.SemaphoreType.DMA,
            pltpu.SemaphoreType.DMA,
        ],
    )(_hist_body)
    return run(he)


def _dense_body(x_ref, hist_ref, w_ref, att_ref, out_ref, loss_ref):
    nb = x_ref.shape[0]
    xw4 = jnp.dot(x_ref[...].reshape(nb * NSRC, F), w_ref[:],
                  preferred_element_type=jnp.float32)
    Hc = hist_ref[0] + hist_ref[1]
    Hj = [Hc[j * HE:(j + 1) * HE, :] for j in range(NB)]
    maskj = [h > 0 for h in Hj]
    deg_e = jnp.zeros((HE, 1), jnp.float32)
    for j in range(NB):
        deg_e = deg_e + jnp.sum(Hj[j], axis=1, keepdims=True)
    Bn = jnp.where(deg_e > 0, 1.0 / deg_e, 0.0)
    ones_col = jnp.ones((HE, 1), jnp.float32)
    Dj = [lax.dot_general(h, ones_col, (((0,), (0,)), ((), ())),
                          preferred_element_type=jnp.float32)
          for h in Hj]

    attv = att_ref[0]
    a1 = attv[:, :F]
    a2 = attv[:, F:]

    acc = None
    sum_i = 0.0
    sum_j = 0.0
    for b in range(nb):
        xwj = [xw4[b * NSRC + j * F:b * NSRC + (j + 1) * F, :]
               for j in range(NB)]
        es = jnp.zeros((HE, F), jnp.float32)
        for j in range(NB):
            es = es + jnp.dot(Hj[j], xwj[j], preferred_element_type=jnp.float32)
        z1 = lax.dot_general(es, a2, (((1,), (1,)), ((), ())),
                             preferred_element_type=jnp.float32)
        out_e = jnp.zeros((HE, F), jnp.float32)
        Sj = []
        for j in range(NB):
            y1 = lax.dot_general(a1, xwj[j], (((1,), (1,)), ((), ())),
                                 preferred_element_type=jnp.float32)
            L = z1 + y1
            L = jnp.where(L >= 0, L, 0.2 * L)
            Lm = jnp.where(maskj[j], L, NEG)
            mx = jnp.max(Lm, axis=0, keepdims=True)
            mx = jnp.where(mx > 0.5 * NEG, mx, 0.0)
            CE = Hj[j] * jnp.exp(Lm - mx)
            s = jnp.sum(CE, axis=0, keepdims=True)
            S = CE / (s + 1e-16)
            Sj.append(S)
            out_e = out_e + jnp.dot(S, xwj[j], preferred_element_type=jnp.float32)
        out_e = Bn * out_e
        for j in range(NB):
            out_n = Dj[j] * lax.dot_general(
                Sj[j], out_e, (((0,), (0,)), ((), ())),
                preferred_element_type=jnp.float32)
            out_ref[b, j * F:(j + 1) * F, :] = out_n
            sum_i = sum_i + jnp.sum(Dj[j] * jnp.sum(xwj[j], axis=1,
                                                    keepdims=True))
        sum_j = sum_j + jnp.sum(deg_e * jnp.sum(es, axis=1, keepdims=True))

        inner = lax.dot_general(es, es, (((1,), (1,)), ((), ())),
                                preferred_element_type=jnp.float32)
        sq = jnp.sum(es * es, axis=1, keepdims=True)
        norms = jnp.sqrt(sq)
        alpha = inner / (norms * jnp.transpose(norms) + 1e-16)
        d2 = jnp.clip(sq + jnp.transpose(sq) - 2.0 * inner, 0.0, None)
        dist = jnp.sqrt(d2 + 1e-12)
        li = alpha * dist + (1.0 - alpha) * jnp.clip(4.2 - dist, 0.0, None)
        acc = li if acc is None else acc + li

    out_ref[:, NSRC:, :] = jnp.zeros_like(out_ref[:, NSRC:, :])

    mkm = acc * (1.0 / nb)
    row_id = lax.broadcasted_iota(jnp.int32, (HE, 1), 0).astype(jnp.float32)
    num_he = jnp.max(jnp.where(deg_e > 0, row_id + 1.0, 0.0))
    rk = lax.broadcasted_iota(jnp.int32, (HE, HE), 0).astype(jnp.float32)
    rm = lax.broadcasted_iota(jnp.int32, (HE, HE), 1).astype(jnp.float32)
    mask_km = jnp.where((rk < num_he) & (rm < num_he), 1.0, 0.0)
    loss_hyper = jnp.sum(jnp.abs(mkm) * mask_km) / (num_he + 1.0) ** 2
    total = float(NNZ) * nb * F
    loss_ref[0, 0] = jnp.abs((sum_i - sum_j) / total) + loss_hyper


def _dense_call(x, hist, weight, att):
    B, N, _ = x.shape
    return pl.pallas_call(
        _dense_body,
        grid=(1,),
        in_specs=[
            pl.BlockSpec((B, NSRC, F), lambda b: (0, 0, 0)),
            pl.BlockSpec((2, NB * HE, F), lambda b: (0, 0, 0)),
            pl.BlockSpec((F, F), lambda b: (0, 0)),
            pl.BlockSpec((1, 1, 2 * F), lambda b: (0, 0, 0)),
        ],
        out_specs=[
            pl.BlockSpec((B, N, F), lambda b: (0, 0, 0)),
            pl.BlockSpec(memory_space=pltpu.SMEM, block_shape=(1, 1),
                         index_map=lambda b: (0, 0)),
        ],
        out_shape=[
            jax.ShapeDtypeStruct((B, N, F), jnp.float32),
            jax.ShapeDtypeStruct((1, 1), jnp.float32),
        ],
    )(x, hist, weight, att)


def kernel(x, hyperedge_index, weight, att):
    hist = _hist_sc(hyperedge_index).reshape(2, NB * HE, F)
    out, loss = _dense_call(x, hist, weight, att)
    return out, loss[0, 0]


# speedup vs baseline: 1.0230x; 1.0007x over previous
import functools

import jax
import jax.numpy as jnp
from jax import lax
from jax.experimental import pallas as pl
from jax.experimental.pallas import tpu as pltpu
from jax.experimental.pallas import tpu_sc as plsc

HE = 512
NSRC = 512
F = 128
NB = 4
NEG = -1e30

NNZ = 16384
NTILES = 32
CHUNK = NNZ // NTILES
BINS = HE * NSRC
SLICE = BINS // 16
ZCHUNK = 2048


def _hist_body(he_hbm, out_hbm, sv, dv, idxv, onesv, zbuf, shared,
               sem1, sem2, sem3):
    c = lax.axis_index("c")
    s = lax.axis_index("s")
    wid = s * 2 + c
    base = wid * CHUNK
    ld1 = pltpu.async_copy(he_hbm.at[0, pl.ds(base, CHUNK)], sv, sem1)
    ld2 = pltpu.async_copy(he_hbm.at[1, pl.ds(base, CHUNK)], dv, sem2)

    for k in range(ZCHUNK // 16):
        zbuf[pl.ds(k * 16, 16)] = jnp.zeros((16,), jnp.float32)
    zcps = [pltpu.async_copy(
        zbuf, shared.at[pl.ds(s * SLICE + q * ZCHUNK, ZCHUNK)], sem3)
        for q in range(SLICE // ZCHUNK)]
    for k in range(8):
        onesv[pl.ds(k * 16, 16)] = jnp.ones((16,), jnp.float32)
    ld1.wait()
    ld2.wait()
    for k in range(CHUNK // 16):
        srcv = sv[pl.ds(k * 16, 16)]
        dstv = dv[pl.ds(k * 16, 16)]
        f = ((srcv >> 7) << 16) + dstv * 128 + (srcv & 127)
        idxv[k // 8, pl.ds((k % 8) * 16, 16)] = f
    for cp in zcps:
        cp.wait()
    plsc.subcore_barrier()
    cps = [pltpu.async_copy(onesv, shared.at[idxv.at[j]], sem1, add=True)
           for j in range(CHUNK // 128)]
    for cp in cps:
        cp.wait()
    plsc.subcore_barrier()
    off = c * BINS + s * SLICE
    pltpu.sync_copy(shared.at[pl.ds(s * SLICE, SLICE)],
                    out_hbm.at[pl.ds(off, SLICE)])


def _hist_sc(he):
    run = functools.partial(
        pl.kernel,
        mesh=plsc.VectorSubcoreMesh(core_axis_name="c", subcore_axis_name="s"),
        out_type=jax.ShapeDtypeStruct((2 * BINS,), jnp.float32),
        scratch_types=[
            pltpu.VMEM((CHUNK,), jnp.int32),
            pltpu.VMEM((CHUNK,), jnp.int32),
            pltpu.VMEM((CHUNK // 128, 128), jnp.int32),
            pltpu.VMEM((128,), jnp.float32),
            pltpu.VMEM((ZCHUNK,), jnp.float32),
            pltpu.VMEM_SHARED((BINS,), jnp.float32),
            pltpu.SemaphoreType.DMA,
            pltpu.SemaphoreType.DMA,
            pltpu.SemaphoreType.DMA,
        ],
    )(_hist_body)
    return run(he)


def _proj_body(x_ref, w_ref, xw_ref):
    nb = x_ref.shape[0]
    xw_ref[...] = jnp.dot(x_ref[...].reshape(nb * NSRC, F), w_ref[:],
                          preferred_element_type=jnp.float32)


def _proj_call(x, weight):
    B = x.shape[0]
    return pl.pallas_call(
        _proj_body,
        grid=(1,),
        in_specs=[
            pl.BlockSpec((B, NSRC, F), lambda b: (0, 0, 0)),
            pl.BlockSpec((F, F), lambda b: (0, 0)),
        ],
        out_specs=pl.BlockSpec((B * NSRC, F), lambda b: (0, 0)),
        out_shape=jax.ShapeDtypeStruct((B * NSRC, F), jnp.float32),
    )(x, weight)


def _dense_body(xw_ref, hist_ref, att_ref, out_ref, loss_ref):
    nb = xw_ref.shape[0] // NSRC
    xw4 = xw_ref[...]
    Hc = hist_ref[0] + hist_ref[1]
    Hj = [Hc[j * HE:(j + 1) * HE, :] for j in range(NB)]
    maskj = [h > 0 for h in Hj]
    deg_e = jnp.zeros((HE, 1), jnp.float32)
    for j in range(NB):
        deg_e = deg_e + jnp.sum(Hj[j], axis=1, keepdims=True)
    Bn = jnp.where(deg_e > 0, 1.0 / deg_e, 0.0)
    ones_col = jnp.ones((HE, 1), jnp.float32)
    Dj = [lax.dot_general(h, ones_col, (((0,), (0,)), ((), ())),
                          preferred_element_type=jnp.float32)
          for h in Hj]

    attv = att_ref[0]
    a1 = attv[:, :F]
    a2 = attv[:, F:]

    acc = None
    sum_i = 0.0
    sum_j = 0.0
    for b in range(nb):
        xwj = [xw4[b * NSRC + j * F:b * NSRC + (j + 1) * F, :]
               for j in range(NB)]
        es = jnp.zeros((HE, F), jnp.float32)
        for j in range(NB):
            es = es + jnp.dot(Hj[j], xwj[j], preferred_element_type=jnp.float32)
        z1 = lax.dot_general(es, a2, (((1,), (1,)), ((), ())),
                             preferred_element_type=jnp.float32)
        out_e = jnp.zeros((HE, F), jnp.float32)
        Sj = []
        for j in range(NB):
            y1 = lax.dot_general(a1, xwj[j], (((1,), (1,)), ((), ())),
                                 preferred_element_type=jnp.float32)
            L = z1 + y1
            L = jnp.where(L >= 0, L, 0.2 * L)
            Lm = jnp.where(maskj[j], L, NEG)
            mx = jnp.max(Lm, axis=0, keepdims=True)
            mx = jnp.where(mx > 0.5 * NEG, mx, 0.0)
            CE = Hj[j] * jnp.exp(Lm - mx)
            s = jnp.sum(CE, axis=0, keepdims=True)
            S = CE / (s + 1e-16)
            Sj.append(S)
            out_e = out_e + jnp.dot(S, xwj[j], preferred_element_type=jnp.float32)
        out_e = Bn * out_e
        for j in range(NB):
            out_n = Dj[j] * lax.dot_general(
                Sj[j], out_e, (((0,), (0,)), ((), ())),
                preferred_element_type=jnp.float32)
            out_ref[b, j * F:(j + 1) * F, :] = out_n
            sum_i = sum_i + jnp.sum(Dj[j] * jnp.sum(xwj[j], axis=1,
                                                    keepdims=True))
        sum_j = sum_j + jnp.sum(deg_e * jnp.sum(es, axis=1, keepdims=True))

        inner = lax.dot_general(es, es, (((1,), (1,)), ((), ())),
                                preferred_element_type=jnp.float32)
        sq = jnp.sum(es * es, axis=1, keepdims=True)
        norms = jnp.sqrt(sq)
        alpha = inner / (norms * jnp.transpose(norms) + 1e-16)
        d2 = jnp.clip(sq + jnp.transpose(sq) - 2.0 * inner, 0.0, None)
        dist = jnp.sqrt(d2 + 1e-12)
        li = alpha * dist + (1.0 - alpha) * jnp.clip(4.2 - dist, 0.0, None)
        acc = li if acc is None else acc + li

    out_ref[:, NSRC:, :] = jnp.zeros_like(out_ref[:, NSRC:, :])

    mkm = acc * (1.0 / nb)
    row_id = lax.broadcasted_iota(jnp.int32, (HE, 1), 0).astype(jnp.float32)
    num_he = jnp.max(jnp.where(deg_e > 0, row_id + 1.0, 0.0))
    rk = lax.broadcasted_iota(jnp.int32, (HE, HE), 0).astype(jnp.float32)
    rm = lax.broadcasted_iota(jnp.int32, (HE, HE), 1).astype(jnp.float32)
    mask_km = jnp.where((rk < num_he) & (rm < num_he), 1.0, 0.0)
    loss_hyper = jnp.sum(jnp.abs(mkm) * mask_km) / (num_he + 1.0) ** 2
    total = float(NNZ) * nb * F
    loss_ref[0, 0] = jnp.abs((sum_i - sum_j) / total) + loss_hyper


def _dense_call(xw, hist, att, B, N):
    return pl.pallas_call(
        _dense_body,
        grid=(1,),
        in_specs=[
            pl.BlockSpec((B * NSRC, F), lambda b: (0, 0)),
            pl.BlockSpec((2, NB * HE, F), lambda b: (0, 0, 0)),
            pl.BlockSpec((1, 1, 2 * F), lambda b: (0, 0, 0)),
        ],
        out_specs=[
            pl.BlockSpec((B, N, F), lambda b: (0, 0, 0)),
            pl.BlockSpec(memory_space=pltpu.SMEM, block_shape=(1, 1),
                         index_map=lambda b: (0, 0)),
        ],
        out_shape=[
            jax.ShapeDtypeStruct((B, N, F), jnp.float32),
            jax.ShapeDtypeStruct((1, 1), jnp.float32),
        ],
    )(xw, hist, att)


def kernel(x, hyperedge_index, weight, att):
    hist = _hist_sc(hyperedge_index).reshape(2, NB * HE, F)
    xw = _proj_call(x, weight)
    out, loss = _dense_call(xw, hist, att, x.shape[0], x.shape[1])
    return out, loss[0, 0]


# speedup vs baseline: 1.0248x; 1.0018x over previous
import functools

import jax
import jax.numpy as jnp
from jax import lax
from jax.experimental import pallas as pl
from jax.experimental.pallas import tpu as pltpu
from jax.experimental.pallas import tpu_sc as plsc

HE = 512
NSRC = 512
F = 128
NB = 4
NEG = -1e30

NNZ = 16384
NTILES = 32
CHUNK = NNZ // NTILES
BINS = HE * NSRC
SLICE = BINS // 16
ZCHUNK = 2048


def _hist_body(he_hbm, out_hbm, sv, dv, idxv, onesv, zbuf, shared,
               sem1, sem2, sem3):
    c = lax.axis_index("c")
    s = lax.axis_index("s")
    wid = s * 2 + c
    base = wid * CHUNK
    ld1 = pltpu.async_copy(he_hbm.at[0, pl.ds(base, CHUNK)], sv, sem1)
    ld2 = pltpu.async_copy(he_hbm.at[1, pl.ds(base, CHUNK)], dv, sem2)

    for k in range(ZCHUNK // 16):
        zbuf[pl.ds(k * 16, 16)] = jnp.zeros((16,), jnp.float32)
    zcps = [pltpu.async_copy(
        zbuf, shared.at[pl.ds(s * SLICE + q * ZCHUNK, ZCHUNK)], sem3)
        for q in range(SLICE // ZCHUNK)]
    for k in range(8):
        onesv[pl.ds(k * 16, 16)] = jnp.ones((16,), jnp.float32)
    ld1.wait()
    ld2.wait()
    for k in range(CHUNK // 16):
        srcv = sv[pl.ds(k * 16, 16)]
        dstv = dv[pl.ds(k * 16, 16)]
        f = ((srcv >> 7) << 16) + dstv * 128 + (srcv & 127)
        idxv[k // 8, pl.ds((k % 8) * 16, 16)] = f
    for cp in zcps:
        cp.wait()
    plsc.subcore_barrier()
    cps = [pltpu.async_copy(onesv, shared.at[idxv.at[j]], sem1, add=True)
           for j in range(CHUNK // 128)]
    for cp in cps:
        cp.wait()
    plsc.subcore_barrier()
    off = c * BINS + s * SLICE
    pltpu.sync_copy(shared.at[pl.ds(s * SLICE, SLICE)],
                    out_hbm.at[pl.ds(off, SLICE)])


def _hist_sc(he):
    run = functools.partial(
        pl.kernel,
        mesh=plsc.VectorSubcoreMesh(core_axis_name="c", subcore_axis_name="s"),
        out_type=jax.ShapeDtypeStruct((2 * BINS,), jnp.float32),
        scratch_types=[
            pltpu.VMEM((CHUNK,), jnp.int32),
            pltpu.VMEM((CHUNK,), jnp.int32),
            pltpu.VMEM((CHUNK // 128, 128), jnp.int32),
            pltpu.VMEM((128,), jnp.float32),
            pltpu.VMEM((ZCHUNK,), jnp.float32),
            pltpu.VMEM_SHARED((BINS,), jnp.float32),
            pltpu.SemaphoreType.DMA,
            pltpu.SemaphoreType.DMA,
            pltpu.SemaphoreType.DMA,
        ],
    )(_hist_body)
    return run(he)


def _dense_body(x_ref, hist_ref, w_ref, att_ref, out_ref, loss_ref):
    nb = x_ref.shape[0]
    xw4 = jnp.dot(x_ref[...].reshape(nb * NSRC, F), w_ref[:],
                  preferred_element_type=jnp.float32)
    Hc = hist_ref[0] + hist_ref[1]
    Hj = [Hc[j * HE:(j + 1) * HE, :] for j in range(NB)]
    maskj = [h > 0 for h in Hj]
    deg_e = jnp.zeros((HE, 1), jnp.float32)
    for j in range(NB):
        deg_e = deg_e + jnp.sum(Hj[j], axis=1, keepdims=True)
    Bn = jnp.where(deg_e > 0, 1.0 / deg_e, 0.0)
    ones_col = jnp.ones((HE, 1), jnp.float32)
    Dj = [lax.dot_general(h, ones_col, (((0,), (0,)), ((), ())),
                          preferred_element_type=jnp.float32)
          for h in Hj]

    attv = att_ref[0]
    a1 = attv[:, :F]
    a2 = attv[:, F:]

    acc = None
    sum_i = 0.0
    sum_j = 0.0
    for b in range(nb):
        xwj = [xw4[b * NSRC + j * F:b * NSRC + (j + 1) * F, :]
               for j in range(NB)]
        es = jnp.zeros((HE, F), jnp.float32)
        for j in range(NB):
            es = es + jnp.dot(Hj[j], xwj[j], preferred_element_type=jnp.float32)
        z1 = lax.dot_general(es, a2, (((1,), (1,)), ((), ())),
                             preferred_element_type=jnp.float32)
        out_e = jnp.zeros((HE, F), jnp.float32)
        Sj = []
        for j in range(NB):
            y1 = lax.dot_general(a1, xwj[j], (((1,), (1,)), ((), ())),
                                 preferred_element_type=jnp.float32)
            L = z1 + y1
            L = jnp.where(L >= 0, L, 0.2 * L)
            Lm = jnp.where(maskj[j], L, NEG)
            mx = jnp.max(Lm, axis=0, keepdims=True)
            mx = jnp.where(mx > 0.5 * NEG, mx, 0.0)
            CE = Hj[j] * jnp.exp(Lm - mx)
            s = jnp.sum(CE, axis=0, keepdims=True)
            S = CE / (s + 1e-16)
            Sj.append(S)
            out_e = out_e + jnp.dot(S, xwj[j], preferred_element_type=jnp.float32)
        out_e = Bn * out_e
        for j in range(NB):
            out_n = Dj[j] * lax.dot_general(
                Sj[j], out_e, (((0,), (0,)), ((), ())),
                preferred_element_type=jnp.float32)
            out_ref[b, j * F:(j + 1) * F, :] = out_n
            sum_i = sum_i + jnp.sum(Dj[j] * jnp.sum(xwj[j], axis=1,
                                                    keepdims=True))
        sum_j = sum_j + jnp.sum(deg_e * jnp.sum(es, axis=1, keepdims=True))

        inner = lax.dot_general(es, es, (((1,), (1,)), ((), ())),
                                preferred_element_type=jnp.float32)
        sq = jnp.sum(es * es, axis=1, keepdims=True)
        norms = jnp.sqrt(sq)
        alpha = inner / (norms * jnp.transpose(norms) + 1e-16)
        d2 = jnp.clip(sq + jnp.transpose(sq) - 2.0 * inner, 0.0, None)
        dist = jnp.sqrt(d2 + 1e-12)
        li = alpha * dist + (1.0 - alpha) * jnp.clip(4.2 - dist, 0.0, None)
        acc = li if acc is None else acc + li

    out_ref[:, NSRC:, :] = jnp.zeros_like(out_ref[:, NSRC:, :])

    mkm = acc * (1.0 / nb)
    row_id = lax.broadcasted_iota(jnp.int32, (HE, 1), 0).astype(jnp.float32)
    num_he = jnp.max(jnp.where(deg_e > 0, row_id + 1.0, 0.0))
    rk = lax.broadcasted_iota(jnp.int32, (HE, HE), 0).astype(jnp.float32)
    rm = lax.broadcasted_iota(jnp.int32, (HE, HE), 1).astype(jnp.float32)
    mask_km = jnp.where((rk < num_he) & (rm < num_he), 1.0, 0.0)
    loss_hyper = jnp.sum(jnp.abs(mkm) * mask_km) / (num_he + 1.0) ** 2
    total = float(NNZ) * nb * F
    loss_ref[0, 0] = jnp.abs((sum_i - sum_j) / total) + loss_hyper


def _dense_call(x, hist, weight, att):
    B, N, _ = x.shape
    return pl.pallas_call(
        _dense_body,
        grid=(1,),
        in_specs=[
            pl.BlockSpec((B, NSRC, F), lambda b: (0, 0, 0)),
            pl.BlockSpec((2, NB * HE, F), lambda b: (0, 0, 0)),
            pl.BlockSpec((F, F), lambda b: (0, 0)),
            pl.BlockSpec((1, 1, 2 * F), lambda b: (0, 0, 0)),
        ],
        out_specs=[
            pl.BlockSpec((B, N, F), lambda b: (0, 0, 0)),
            pl.BlockSpec(memory_space=pltpu.SMEM, block_shape=(1, 1),
                         index_map=lambda b: (0, 0)),
        ],
        out_shape=[
            jax.ShapeDtypeStruct((B, N, F), jnp.float32),
            jax.ShapeDtypeStruct((1, 1), jnp.float32),
        ],
    )(x, hist, weight, att)


def kernel(x, hyperedge_index, weight, att):
    hist = _hist_sc(hyperedge_index).reshape(2, NB * HE, F)
    out, loss = _dense_call(x, hist, weight, att)
    return out, loss[0, 0]
